# Initial kernel scaffold; baseline (speedup 1.0000x reference)
import jax
import jax.numpy as jnp
from jax.experimental import pallas as pl


def kernel(user_emb, item_emb, adj_values, adj_indices):
    raise NotImplementedError("write your pallas kernel here")


# speedup vs baseline: 1.7809x; 1.7809x over previous
import jax
import jax.numpy as jnp
from jax import lax
from jax.experimental import pallas as pl
from jax.experimental.pallas import tpu as pltpu
from jax.experimental.pallas import tpu_sc as plsc

USER_N = 25000
ITEM_N = 25000
NODES = USER_N + ITEM_N
EDGES = 800000
D = 64
PD = 2 * D
P = NODES // 2
NC = 2
NS = 16

HP0 = 12512
TRASH = 12512
ACC2 = 12544
ZCH = 8
CH = 80
EPT = EDGES // NS
NCHUNK = EPT // CH
WCH = 8

MCH = 5120
NMCH = NODES * D // MCH

_mesh = plsc.VectorSubcoreMesh(
    core_axis_name="c", subcore_axis_name="s", num_cores=NC, num_subcores=NS
)


def _spmm_body(ego2, src_h, dst_h, val_h, out, acc, src_v, dst_v, lidx_v,
               gidx_v, val_v, rows_v, msg_v, zbuf, wbuf, sem):
    cid = lax.axis_index("c")
    sid = lax.axis_index("s")
    base_pair = cid * HP0
    npairs = HP0 - cid * (HP0 - (P - HP0))

    z16 = jnp.zeros((16,), jnp.float32)
    for r in range(ZCH):
        for c in range(PD // 16):
            zbuf[r, pl.ds(c * 16, 16)] = z16
    zrows = ACC2 // NS

    def zero_body(i, carry):
        pltpu.sync_copy(zbuf, acc.at[pl.ds(sid * zrows + i * ZCH, ZCH)])
        return carry

    lax.fori_loop(0, zrows // ZCH, zero_body, 0)
    plsc.subcore_barrier()

    t_base = sid * EPT

    def chunk_body(i, carry):
        eb = t_base + i * CH
        pltpu.sync_copy(src_h.at[pl.ds(eb, CH)], src_v)
        pltpu.sync_copy(dst_h.at[pl.ds(eb, CH)], dst_v)
        pltpu.sync_copy(val_h.at[pl.ds(eb, CH)], val_v)

        def group_body(j, carry2):
            off = j * 16
            dp = dst_v[pl.ds(off, 16)] >> 1
            ok = (dp >= base_pair) & (dp < base_pair + npairs)
            lidx_v[pl.ds(off, 16)] = jnp.where(ok, dp - base_pair, TRASH)
            gidx_v[pl.ds(off, 16)] = src_v[pl.ds(off, 16)] >> 1
            return carry2

        lax.fori_loop(0, CH // 16, group_body, 0)
        pltpu.async_copy(ego2.at[gidx_v], rows_v, sem).wait()

        def scale_body(j, carry2):
            off = j * 16
            v16 = val_v[pl.ds(off, 16)]
            sodd = (src_v[pl.ds(off, 16)] & 1).astype(jnp.float32)
            dodd = (dst_v[pl.ds(off, 16)] & 1).astype(jnp.float32)
            for k in range(16):
                ev = v16[k]
                sf = sodd[k]
                df = dodd[k]
                e = off + k
                for c in range(D // 16):
                    lo = rows_v[e, pl.ds(c * 16, 16)]
                    hi = rows_v[e, pl.ds(D + c * 16, 16)]
                    m = (lo + (hi - lo) * sf) * ev
                    msg_v[e, pl.ds(c * 16, 16)] = m * (1.0 - df)
                    msg_v[e, pl.ds(D + c * 16, 16)] = m * df
            return carry2

        lax.fori_loop(0, CH // 16, scale_body, 0)
        pltpu.sync_copy(msg_v, acc.at[lidx_v], add=True)
        return carry

    lax.fori_loop(0, NCHUNK, chunk_body, 0)
    plsc.subcore_barrier()

    nch = npairs // WCH
    nw = nch // NS + jnp.where(sid < nch % NS, 1, 0)

    def w_body(j, carry):
        r0 = (sid + NS * j) * WCH
        pltpu.sync_copy(acc.at[pl.ds(r0, WCH)], wbuf)
        pltpu.sync_copy(wbuf, out.at[pl.ds(base_pair + r0, WCH)])
        return carry

    lax.fori_loop(0, nw, w_body, 0)


_spmm = pl.kernel(
    _spmm_body,
    out_type=jax.ShapeDtypeStruct((P, PD), jnp.float32),
    mesh=_mesh,
    scratch_types=[
        pltpu.VMEM_SHARED((ACC2, PD), jnp.float32),
        pltpu.VMEM((CH,), jnp.int32),
        pltpu.VMEM((CH,), jnp.int32),
        pltpu.VMEM((CH,), jnp.int32),
        pltpu.VMEM((CH,), jnp.int32),
        pltpu.VMEM((CH,), jnp.float32),
        pltpu.VMEM((CH, PD), jnp.float32),
        pltpu.VMEM((CH, PD), jnp.float32),
        pltpu.VMEM((ZCH, PD), jnp.float32),
        pltpu.VMEM((WCH, PD), jnp.float32),
        pltpu.SemaphoreType.DMA,
    ],
)


def _mean_body(l1, l2, l3, out, b1, b2, b3, sem):
    w = lax.axis_index("s") * NC + lax.axis_index("c")

    def body(j, carry):
        o = (w + NC * NS * j) * MCH
        pltpu.sync_copy(l1.at[pl.ds(o, MCH)], b1)
        pltpu.sync_copy(l2.at[pl.ds(o, MCH)], b2)
        pltpu.sync_copy(l3.at[pl.ds(o, MCH)], b3)

        def g_body(g, c2):
            s = pl.ds(g * 16, 16)
            b1[s] = (b1[s] + b2[s] + b3[s]) * jnp.float32(1.0 / 3.0)
            return c2

        lax.fori_loop(0, MCH // 16, g_body, 0)
        pltpu.sync_copy(b1, out.at[pl.ds(o, MCH)])
        return carry

    nw = NMCH // (NC * NS) + jnp.where(w < NMCH % (NC * NS), 1, 0)
    lax.fori_loop(0, nw, body, 0)


_mean = pl.kernel(
    _mean_body,
    out_type=jax.ShapeDtypeStruct((NODES * D,), jnp.float32),
    mesh=_mesh,
    scratch_types=[
        pltpu.VMEM((MCH,), jnp.float32),
        pltpu.VMEM((MCH,), jnp.float32),
        pltpu.VMEM((MCH,), jnp.float32),
        pltpu.SemaphoreType.DMA,
    ],
)


def kernel(user_emb, item_emb, adj_values, adj_indices):
    ego = jnp.concatenate([user_emb, item_emb], axis=0)
    src = adj_indices[0].astype(jnp.int32)
    dst = adj_indices[1].astype(jnp.int32)
    vals = adj_values.astype(jnp.float32)
    l1 = _spmm(ego.reshape(P, PD), src, dst, vals)
    l2 = _spmm(l1, src, dst, vals)
    l3 = _spmm(l2, src, dst, vals)
    flat = _mean(l1.reshape(-1), l2.reshape(-1), l3.reshape(-1))
    final = flat.reshape(NODES, D)
    return final[:USER_N], final[USER_N:]


# speedup vs baseline: 3.0159x; 1.6935x over previous
import jax
import jax.numpy as jnp
from jax import lax
from jax.experimental import pallas as pl
from jax.experimental.pallas import tpu as pltpu
from jax.experimental.pallas import tpu_sc as plsc

USER_N = 25000
ITEM_N = 25000
NODES = USER_N + ITEM_N
EDGES = 800000
D = 64
PD = 2 * D
P = NODES // 2
NC = 2
NS = 16

HP0 = 12512
TRASH = 12512
ACC2 = 12544
ZCH = 8
CH = 80
PKT = 2 * CH
EPT = EDGES // NS
NCHUNK = EPT // CH
WCH = 8

MCH = 5120
NMCH = NODES * D // MCH

_mesh = plsc.VectorSubcoreMesh(
    core_axis_name="c", subcore_axis_name="s", num_cores=NC, num_subcores=NS
)


def _spmm_body(ego2, pkt_h, val_h, out, acc,
               pkt0, pkt1, val0, val1, gidx0, gidx1, lidx0, lidx1,
               rows0, rows1, zbuf, wbuf, sem0, sem1):
    cid = lax.axis_index("c")
    sid = lax.axis_index("s")
    base_pair = cid * HP0
    npairs = HP0 - cid * (HP0 - (P - HP0))

    z16 = jnp.zeros((16,), jnp.float32)
    for r in range(ZCH):
        for c in range(PD // 16):
            zbuf[r, pl.ds(c * 16, 16)] = z16
    zrows = ACC2 // NS

    def zero_body(i, carry):
        pltpu.sync_copy(zbuf, acc.at[pl.ds(sid * zrows + i * ZCH, ZCH)])
        return carry

    lax.fori_loop(0, zrows // ZCH, zero_body, 0)
    plsc.subcore_barrier()

    def prefetch(i, pkt_v, val_v, gidx_v, lidx_v, rows_v, sem):
        gc = sid * NCHUNK + i
        pltpu.sync_copy(pkt_h.at[pl.ds(gc * PKT, PKT)], pkt_v)
        pltpu.sync_copy(val_h.at[pl.ds(gc * CH, CH)], val_v)

        def group_body(j, carry2):
            off = j * 16
            dv = pkt_v[pl.ds(CH + off, 16)]
            dp = dv >> 1
            ok = (dp >= base_pair) & (dp < base_pair + npairs)
            lidx_v[pl.ds(off, 16)] = jnp.where(ok, dp - base_pair, TRASH)
            gidx_v[pl.ds(off, 16)] = pkt_v[pl.ds(off, 16)] >> 1
            return carry2

        lax.fori_loop(0, CH // 16, group_body, 0)
        pltpu.async_copy(ego2.at[gidx_v], rows_v, sem)

    def compute(pkt_v, val_v, gidx_v, lidx_v, rows_v, sem):
        pltpu.make_async_copy(ego2.at[gidx_v], rows_v, sem).wait()

        def scale_body(j, carry2):
            off = j * 16
            v16 = val_v[pl.ds(off, 16)]
            sodd = (pkt_v[pl.ds(off, 16)] & 1).astype(jnp.float32)
            dodd = (pkt_v[pl.ds(CH + off, 16)] & 1).astype(jnp.float32)
            for k in range(16):
                ev = v16[k]
                sf = sodd[k]
                df = dodd[k]
                e = off + k
                for c in range(D // 16):
                    lo = rows_v[e, pl.ds(c * 16, 16)]
                    hi = rows_v[e, pl.ds(D + c * 16, 16)]
                    m = (lo + (hi - lo) * sf) * ev
                    a = m * df
                    rows_v[e, pl.ds(c * 16, 16)] = m - a
                    rows_v[e, pl.ds(D + c * 16, 16)] = a
            return carry2

        lax.fori_loop(0, CH // 16, scale_body, 0)
        pltpu.sync_copy(rows_v, acc.at[lidx_v], add=True)

    prefetch(0, pkt0, val0, gidx0, lidx0, rows0, sem0)

    def chunk_body(j, carry):
        i = 2 * j
        prefetch(i + 1, pkt1, val1, gidx1, lidx1, rows1, sem1)
        compute(pkt0, val0, gidx0, lidx0, rows0, sem0)
        prefetch(i + 2, pkt0, val0, gidx0, lidx0, rows0, sem0)
        compute(pkt1, val1, gidx1, lidx1, rows1, sem1)
        return carry

    lax.fori_loop(0, (NCHUNK - 1) // 2, chunk_body, 0)
    compute(pkt0, val0, gidx0, lidx0, rows0, sem0)
    plsc.subcore_barrier()

    nch = npairs // WCH
    nw = nch // NS + jnp.where(sid < nch % NS, 1, 0)

    def w_body(j, carry):
        r0 = (sid + NS * j) * WCH
        pltpu.sync_copy(acc.at[pl.ds(r0, WCH)], wbuf)
        pltpu.sync_copy(wbuf, out.at[pl.ds(base_pair + r0, WCH)])
        return carry

    lax.fori_loop(0, nw, w_body, 0)


_spmm = pl.kernel(
    _spmm_body,
    out_type=jax.ShapeDtypeStruct((P, PD), jnp.float32),
    mesh=_mesh,
    scratch_types=[
        pltpu.VMEM_SHARED((ACC2, PD), jnp.float32),
        pltpu.VMEM((PKT,), jnp.int32),
        pltpu.VMEM((PKT,), jnp.int32),
        pltpu.VMEM((CH,), jnp.float32),
        pltpu.VMEM((CH,), jnp.float32),
        pltpu.VMEM((CH,), jnp.int32),
        pltpu.VMEM((CH,), jnp.int32),
        pltpu.VMEM((CH,), jnp.int32),
        pltpu.VMEM((CH,), jnp.int32),
        pltpu.VMEM((CH, PD), jnp.float32),
        pltpu.VMEM((CH, PD), jnp.float32),
        pltpu.VMEM((ZCH, PD), jnp.float32),
        pltpu.VMEM((WCH, PD), jnp.float32),
        pltpu.SemaphoreType.DMA,
        pltpu.SemaphoreType.DMA,
    ],
)


def _mean_body(l1, l2, l3, out, b1, b2, b3, sem):
    w = lax.axis_index("s") * NC + lax.axis_index("c")

    def body(j, carry):
        o = (w + NC * NS * j) * MCH
        pltpu.sync_copy(l1.at[pl.ds(o, MCH)], b1)
        pltpu.sync_copy(l2.at[pl.ds(o, MCH)], b2)
        pltpu.sync_copy(l3.at[pl.ds(o, MCH)], b3)

        def g_body(g, c2):
            s = pl.ds(g * 16, 16)
            b1[s] = (b1[s] + b2[s] + b3[s]) * jnp.float32(1.0 / 3.0)
            return c2

        lax.fori_loop(0, MCH // 16, g_body, 0)
        pltpu.sync_copy(b1, out.at[pl.ds(o, MCH)])
        return carry

    nw = NMCH // (NC * NS) + jnp.where(w < NMCH % (NC * NS), 1, 0)
    lax.fori_loop(0, nw, body, 0)


_mean = pl.kernel(
    _mean_body,
    out_type=jax.ShapeDtypeStruct((NODES * D,), jnp.float32),
    mesh=_mesh,
    scratch_types=[
        pltpu.VMEM((MCH,), jnp.float32),
        pltpu.VMEM((MCH,), jnp.float32),
        pltpu.VMEM((MCH,), jnp.float32),
        pltpu.SemaphoreType.DMA,
    ],
)


def kernel(user_emb, item_emb, adj_values, adj_indices):
    ego = jnp.concatenate([user_emb, item_emb], axis=0)
    src = adj_indices[0].astype(jnp.int32)
    dst = adj_indices[1].astype(jnp.int32)
    vals = adj_values.astype(jnp.float32)
    nchunks = EDGES // CH
    pkt = jnp.stack(
        [src.reshape(nchunks, CH), dst.reshape(nchunks, CH)],
        axis=1).reshape(-1)
    l1 = _spmm(ego.reshape(P, PD), pkt, vals)
    l2 = _spmm(l1, pkt, vals)
    l3 = _spmm(l2, pkt, vals)
    flat = _mean(l1.reshape(-1), l2.reshape(-1), l3.reshape(-1))
    final = flat.reshape(NODES, D)
    return final[:USER_N], final[USER_N:]
